# baseline (speedup 1.0000x reference)
import jax
import jax.numpy as jnp
from jax import lax
from jax.experimental import pallas as pl
from jax.experimental.pallas import tpu as pltpu

_BATCH = 4096
_NCARDS = 26
_CARD = 100
_BR = 256


def _onehot_block(x_ref, o_ref):
    pos = lax.broadcasted_iota(jnp.int32, o_ref.shape, 2)
    o_ref[...] = (x_ref[...] == pos).astype(jnp.int32)


def kernel(x):
    x3 = x.reshape(_BATCH, _NCARDS, 1)
    out3 = pl.pallas_call(
        _onehot_block,
        grid=(_BATCH // _BR,),
        in_specs=[pl.BlockSpec((_BR, _NCARDS, 1), lambda r: (r, 0, 0))],
        out_specs=pl.BlockSpec((_BR, _NCARDS, _CARD), lambda r: (r, 0, 0)),
        out_shape=jax.ShapeDtypeStruct((_BATCH, _NCARDS, _CARD), jnp.int32),
        compiler_params=pltpu.CompilerParams(
            dimension_semantics=("parallel",)
        ),
    )(x3)
    return out3.reshape(_BATCH, _NCARDS * _CARD)


# speedup vs baseline: 3.1199x; 3.1199x over previous
import jax
import jax.numpy as jnp
from jax import lax
from jax.experimental import pallas as pl
from jax.experimental.pallas import tpu as pltpu

_BATCH = 4096
_NCARDS = 26
_CARD = 100
_WIDTH = _NCARDS * _CARD
_BR = 256


def _onehot_block(x_ref, r_ref, o_ref):
    xr = jnp.dot(x_ref[...], r_ref[...], preferred_element_type=jnp.float32)
    j = lax.broadcasted_iota(jnp.int32, o_ref.shape, 1)
    pos = (j - (j // _CARD) * _CARD).astype(jnp.float32)
    o_ref[...] = (xr == pos).astype(jnp.int32)


def kernel(x):
    xb = x.astype(jnp.bfloat16)
    card_of_col = jnp.arange(_WIDTH, dtype=jnp.int32) // _CARD
    rep = (card_of_col[None, :] == jnp.arange(_NCARDS, dtype=jnp.int32)[:, None]
           ).astype(jnp.bfloat16)
    return pl.pallas_call(
        _onehot_block,
        grid=(_BATCH // _BR,),
        in_specs=[
            pl.BlockSpec((_BR, _NCARDS), lambda r: (r, 0)),
            pl.BlockSpec((_NCARDS, _WIDTH), lambda r: (0, 0)),
        ],
        out_specs=pl.BlockSpec((_BR, _WIDTH), lambda r: (r, 0)),
        out_shape=jax.ShapeDtypeStruct((_BATCH, _WIDTH), jnp.int32),
        compiler_params=pltpu.CompilerParams(
            dimension_semantics=("parallel",)
        ),
    )(xb, rep)


# speedup vs baseline: 3.3069x; 1.0599x over previous
import jax
import jax.numpy as jnp
from jax import lax
from jax.experimental import pallas as pl
from jax.experimental.pallas import tpu as pltpu

_BATCH = 4096
_NCARDS = 26
_CARD = 100
_WIDTH = _NCARDS * _CARD
_BR = 512


def _onehot_block(x_ref, r_ref, o_ref):
    xr = jnp.dot(x_ref[...], r_ref[...], preferred_element_type=jnp.float32)
    j = lax.broadcasted_iota(jnp.int32, o_ref.shape, 1)
    pos = (j - (j // _CARD) * _CARD).astype(jnp.float32)
    o_ref[...] = (xr == pos).astype(jnp.int32)


def kernel(x):
    xb = x.astype(jnp.bfloat16)
    card_of_col = jnp.arange(_WIDTH, dtype=jnp.int32) // _CARD
    rep = (card_of_col[None, :] == jnp.arange(_NCARDS, dtype=jnp.int32)[:, None]
           ).astype(jnp.bfloat16)
    return pl.pallas_call(
        _onehot_block,
        grid=(_BATCH // _BR,),
        in_specs=[
            pl.BlockSpec((_BR, _NCARDS), lambda r: (r, 0)),
            pl.BlockSpec((_NCARDS, _WIDTH), lambda r: (0, 0)),
        ],
        out_specs=pl.BlockSpec((_BR, _WIDTH), lambda r: (r, 0)),
        out_shape=jax.ShapeDtypeStruct((_BATCH, _WIDTH), jnp.int32),
        compiler_params=pltpu.CompilerParams(
            dimension_semantics=("parallel",)
        ),
    )(xb, rep)
